# Initial kernel scaffold; baseline (speedup 1.0000x reference)
import jax
import jax.numpy as jnp
from jax.experimental import pallas as pl


def kernel(t_list, ent_embeds, edge_index, edge_type, node_ids, node_graph_ids, bases1, coeffs1, wself1, bases2, coeffs2, wself2):
    raise NotImplementedError("write your pallas kernel here")


# speedup vs baseline: 2.9569x; 2.9569x over previous
import functools
import jax
import jax.numpy as jnp
from jax import lax
from jax.experimental import pallas as pl
from jax.experimental.pallas import tpu as pltpu, tpu_sc as plsc

HID = 128
R = 16
NB = 5
SEQ = 10
NG = 20
NPAD = 10240
NC = 2
NS = 16
NW = NC * NS
EK = 128


def _sc_gather_body(table, idx_hbm, out, idx_v, rows_v, sem):
    c = lax.axis_index("c")
    s = lax.axis_index("s")
    wid = c * NS + s
    bpw = NPAD // NW

    def step(i, _):
        off = wid * bpw + i * 64
        pltpu.sync_copy(idx_hbm.at[pl.ds(off, 64)], idx_v)
        pltpu.async_copy(table.at[idx_v], rows_v, sem).wait()
        pltpu.sync_copy(rows_v, out.at[pl.ds(off, 64)])
        return 0

    lax.fori_loop(0, bpw // 64, step, 0)


def _sc_gather(ent_embeds, node_ids_pad):
    k = functools.partial(
        pl.kernel,
        mesh=plsc.VectorSubcoreMesh(core_axis_name="c", subcore_axis_name="s"),
        out_type=jax.ShapeDtypeStruct((NPAD, HID), jnp.float32),
        scratch_types=[
            pltpu.VMEM((64,), jnp.int32),
            pltpu.VMEM((64, HID), jnp.float32),
            pltpu.SemaphoreType.DMA,
        ],
    )(_sc_gather_body)
    return k(ent_embeds, node_ids_pad)


def _sc_edge_body(nchunks, with_deg, xrel, gidx_hbm, dst_hbm, zrows, zdeg,
                  onesrc, *refs):
    if with_deg:
        aggout, degout, gidx_v, dst_v, rows_v, ones_v, agg_sh, deg_sh, sem = refs
    else:
        aggout, gidx_v, dst_v, rows_v, agg_sh, sem = refs
        deg_sh = ones_v = degout = None

    c = lax.axis_index("c")
    s = lax.axis_index("s")
    wid = c * NS + s
    rows = NPAD // NS
    pltpu.sync_copy(zrows, agg_sh.at[pl.ds(s * rows, rows)])
    if with_deg:
        pltpu.sync_copy(zdeg, deg_sh.at[pl.ds(s * rows, rows)])
        pltpu.sync_copy(onesrc, ones_v)
    plsc.subcore_barrier()

    ept = nchunks * EK

    def step(i, _):
        off = wid * ept + i * EK
        pltpu.sync_copy(gidx_hbm.at[pl.ds(off, EK)], gidx_v)
        pltpu.sync_copy(dst_hbm.at[pl.ds(off, EK)], dst_v)
        pltpu.async_copy(xrel.at[gidx_v], rows_v, sem).wait()
        pltpu.sync_copy(rows_v, agg_sh.at[dst_v], add=True)
        if with_deg:
            pltpu.sync_copy(ones_v, deg_sh.at[dst_v], add=True)
        return 0

    lax.fori_loop(0, nchunks, step, 0)
    plsc.subcore_barrier()
    pltpu.sync_copy(agg_sh.at[pl.ds(s * rows, rows)],
                    aggout.at[c, pl.ds(s * rows, rows)])
    if with_deg:
        pltpu.sync_copy(deg_sh.at[pl.ds(s * rows, rows)],
                        degout.at[c, pl.ds(s * rows, rows)])


def _sc_edge(xrel, gidx, dst, with_deg):
    e_pad = gidx.shape[0]
    nchunks = e_pad // (NW * EK)
    zrows = jnp.zeros((NPAD // NS, HID), jnp.float32)
    zdeg = jnp.zeros((NPAD // NS,), jnp.float32)
    onesrc = jnp.ones((EK,), jnp.float32)
    outs = [jax.ShapeDtypeStruct((NC, NPAD, HID), jnp.float32)]
    scratch = [
        pltpu.VMEM((EK,), jnp.int32),
        pltpu.VMEM((EK,), jnp.int32),
        pltpu.VMEM((EK, HID), jnp.float32),
    ]
    if with_deg:
        outs.append(jax.ShapeDtypeStruct((NC, NPAD), jnp.float32))
        scratch.append(pltpu.VMEM((EK,), jnp.float32))
    scratch.append(pltpu.VMEM_SHARED((NPAD, HID), jnp.float32))
    if with_deg:
        scratch.append(pltpu.VMEM_SHARED((NPAD,), jnp.float32))
    scratch.append(pltpu.SemaphoreType.DMA)
    k = functools.partial(
        pl.kernel,
        mesh=plsc.VectorSubcoreMesh(core_axis_name="c", subcore_axis_name="s"),
        out_type=tuple(outs) if with_deg else outs[0],
        scratch_types=scratch,
    )(functools.partial(_sc_edge_body, nchunks, with_deg))
    return k(xrel, gidx, dst, zrows, zdeg, onesrc)


def _tc_xrel_body(x_ref, bases_ref, coeffs_ref, out_ref):
    r = pl.program_id(0)
    sel = (lax.broadcasted_iota(jnp.int32, (R, 1), 0) == r).astype(jnp.float32)
    cvec = jnp.sum(coeffs_ref[...] * sel, axis=0)
    w = jnp.tensordot(cvec, bases_ref[...], axes=[[0], [0]])
    out_ref[0] = jnp.dot(x_ref[...], w, preferred_element_type=jnp.float32)


def _tc_xrel(x, bases, coeffs):
    blk = 512
    out = pl.pallas_call(
        _tc_xrel_body,
        grid=(R, NPAD // blk),
        in_specs=[
            pl.BlockSpec((blk, HID), lambda r, b: (b, 0)),
            pl.BlockSpec((NB, HID, HID), lambda r, b: (0, 0, 0)),
            pl.BlockSpec((R, NB), lambda r, b: (0, 0)),
        ],
        out_specs=pl.BlockSpec((1, blk, HID), lambda r, b: (r, b, 0)),
        out_shape=jax.ShapeDtypeStruct((R, NPAD, HID), jnp.float32),
    )(x, bases, coeffs)
    return out.reshape(R * NPAD, HID)


def _tc_combine_body(aggp_ref, degp_ref, x_ref, wself_ref, out_ref):
    agg = jnp.sum(aggp_ref[...], axis=0)
    deg = jnp.sum(degp_ref[...], axis=0)
    agg = agg / jnp.maximum(deg, 1.0)[:, None]
    out_ref[...] = jax.nn.relu(agg + jnp.dot(x_ref[...], wself_ref[...],
                                             preferred_element_type=jnp.float32))


def _tc_combine(aggp, degp, x, wself):
    blk = 512
    return pl.pallas_call(
        _tc_combine_body,
        grid=(NPAD // blk,),
        in_specs=[
            pl.BlockSpec((NC, blk, HID), lambda b: (0, b, 0)),
            pl.BlockSpec((NC, blk), lambda b: (0, b)),
            pl.BlockSpec((blk, HID), lambda b: (b, 0)),
            pl.BlockSpec((HID, HID), lambda b: (0, 0)),
        ],
        out_specs=pl.BlockSpec((blk, HID), lambda b: (b, 0)),
        out_shape=jax.ShapeDtypeStruct((NPAD, HID), jnp.float32),
    )(aggp, degp, x, wself)


def _tc_pool_body(h_ref, gid_ref, t_ref, out_ref, acc_ref):
    i = pl.program_id(0)
    nblocks = pl.num_programs(0)

    @pl.when(i == 0)
    def _():
        acc_ref[...] = jnp.full((32, HID), -1.0, jnp.float32)

    gid = gid_ref[...]
    hb = h_ref[...]
    for g in range(NG):
        m = gid == g
        vals = jnp.where(m, hb, -1.0)
        mx = jnp.max(vals, axis=0)
        acc_ref[g, :] = jnp.maximum(acc_ref[g, :], mx)

    @pl.when(i == nblocks - 1)
    def _():
        gi = acc_ref[...]
        gi = jnp.where(gi < 0.0, 0.0, gi)
        trep = t_ref[...]
        nrow = trep.shape[0]
        j2 = lax.broadcasted_iota(jnp.int32, (nrow, 1), 0) % SEQ
        start = jnp.maximum(trep - SEQ, 0)
        tsel = jnp.clip(start + j2, 0, NG - 1)
        msk = (j2 < (trep - start)).astype(jnp.float32)
        onehot = (lax.broadcasted_iota(jnp.int32, (nrow, 32), 1)
                  == tsel).astype(jnp.float32)
        res = jnp.dot(onehot, gi, preferred_element_type=jnp.float32)
        out_ref[...] = res * msk


def _tc_pool(h, gid_pad, t_rep):
    blk = 512
    nrow = t_rep.shape[0]
    return pl.pallas_call(
        _tc_pool_body,
        grid=(NPAD // blk,),
        in_specs=[
            pl.BlockSpec((blk, HID), lambda i: (i, 0)),
            pl.BlockSpec((blk, 1), lambda i: (i, 0)),
            pl.BlockSpec((nrow, 1), lambda i: (0, 0)),
        ],
        out_specs=pl.BlockSpec((nrow, HID), lambda i: (0, 0)),
        out_shape=jax.ShapeDtypeStruct((nrow, HID), jnp.float32),
        scratch_shapes=[pltpu.VMEM((32, HID), jnp.float32)],
    )(h, gid_pad, t_rep)


def kernel(t_list, ent_embeds, edge_index, edge_type, node_ids, node_graph_ids,
           bases1, coeffs1, wself1, bases2, coeffs2, wself2):
    n = node_ids.shape[0]
    e = edge_type.shape[0]
    nid_pad = jnp.concatenate(
        [node_ids, jnp.zeros((NPAD - n,), jnp.int32)])
    gid_pad = jnp.concatenate(
        [node_graph_ids, jnp.full((NPAD - n,), 31, jnp.int32)])[:, None]
    src2 = jnp.concatenate([edge_index[0], edge_index[1]])
    dst2 = jnp.concatenate([edge_index[1], edge_index[0]])
    et2 = jnp.concatenate([edge_type, edge_type])
    e2 = 2 * e
    grain = NW * EK
    e_pad = ((e2 + grain - 1) // grain) * grain
    gidx = et2 * NPAD + src2
    gidx = jnp.concatenate([gidx, jnp.zeros((e_pad - e2,), jnp.int32)])
    dstp = jnp.concatenate(
        [dst2, jnp.full((e_pad - e2,), n, jnp.int32)])

    x = _sc_gather(ent_embeds, nid_pad)
    xrel1 = _tc_xrel(x, bases1, coeffs1)
    aggp1, degp = _sc_edge(xrel1, gidx, dstp, with_deg=True)
    h1 = _tc_combine(aggp1, degp, x, wself1)
    xrel2 = _tc_xrel(h1, bases2, coeffs2)
    aggp2 = _sc_edge(xrel2, gidx, dstp, with_deg=False)
    h2 = _tc_combine(aggp2, degp, h1, wself2)
    emb = _tc_pool(h2, gid_pad, jnp.repeat(t_list, SEQ)[:, None])
    return emb.reshape(t_list.shape[0], SEQ, HID)


# speedup vs baseline: 3.0605x; 1.0350x over previous
import functools
import jax
import jax.numpy as jnp
from jax import lax
from jax.experimental import pallas as pl
from jax.experimental.pallas import tpu as pltpu, tpu_sc as plsc

HID = 128
R = 16
NB = 5
SEQ = 10
NG = 20
NPAD = 10240
NC = 2
NS = 16
NW = NC * NS
EK = 128


def _sc_gather_body(table, idx_hbm, out, idx_v, rows_v, sem):
    c = lax.axis_index("c")
    s = lax.axis_index("s")
    wid = c * NS + s
    bpw = NPAD // NW

    def step(i, _):
        off = wid * bpw + i * 64
        pltpu.sync_copy(idx_hbm.at[pl.ds(off, 64)], idx_v)
        pltpu.async_copy(table.at[idx_v], rows_v, sem).wait()
        pltpu.sync_copy(rows_v, out.at[pl.ds(off, 64)])
        return 0

    lax.fori_loop(0, bpw // 64, step, 0)


def _sc_gather(ent_embeds, node_ids_pad):
    k = functools.partial(
        pl.kernel,
        mesh=plsc.VectorSubcoreMesh(core_axis_name="c", subcore_axis_name="s"),
        out_type=jax.ShapeDtypeStruct((NPAD, HID), jnp.float32),
        scratch_types=[
            pltpu.VMEM((64,), jnp.int32),
            pltpu.VMEM((64, HID), jnp.float32),
            pltpu.SemaphoreType.DMA,
        ],
    )(_sc_gather_body)
    return k(ent_embeds, node_ids_pad)


def _sc_edge_body(nchunks, with_deg, xrel, gidx_hbm, dst_hbm, zrows, zdeg,
                  onesrc, *refs):
    if with_deg:
        (aggout, degout, gidx_a, dst_a, rows_a, gidx_b, dst_b, rows_b,
         ones_v, agg_sh, deg_sh, sema, semb) = refs
    else:
        (aggout, gidx_a, dst_a, rows_a, gidx_b, dst_b, rows_b,
         agg_sh, sema, semb) = refs
        deg_sh = ones_v = degout = None

    c = lax.axis_index("c")
    s = lax.axis_index("s")
    wid = c * NS + s
    rows = NPAD // NS
    pltpu.sync_copy(zrows, agg_sh.at[pl.ds(s * rows, rows)])
    if with_deg:
        pltpu.sync_copy(zdeg, deg_sh.at[pl.ds(s * rows, rows)])
        pltpu.sync_copy(onesrc, ones_v)
    plsc.subcore_barrier()

    ept = nchunks * EK
    npairs = nchunks // 2

    def fire(i, gidx_v, dst_v, rows_v, sem):
        off = wid * ept + i * EK
        pltpu.sync_copy(gidx_hbm.at[pl.ds(off, EK)], gidx_v)
        pltpu.sync_copy(dst_hbm.at[pl.ds(off, EK)], dst_v)
        return pltpu.async_copy(xrel.at[gidx_v], rows_v, sem)

    def drain(rows_v, sem):
        pltpu.make_async_copy(xrel.at[pl.ds(0, EK)], rows_v, sem).wait()

    def scatter(dst_v, rows_v):
        pltpu.sync_copy(rows_v, agg_sh.at[dst_v], add=True)
        if with_deg:
            pltpu.sync_copy(ones_v, deg_sh.at[dst_v], add=True)

    fire(0, gidx_a, dst_a, rows_a, sema)

    def pair(p, _):
        hb = fire(2 * p + 1, gidx_b, dst_b, rows_b, semb)
        drain(rows_a, sema)
        scatter(dst_a, rows_a)

        @pl.when(p < npairs - 1)
        def _():
            fire(2 * p + 2, gidx_a, dst_a, rows_a, sema)

        hb.wait()
        scatter(dst_b, rows_b)
        return 0

    lax.fori_loop(0, npairs, pair, 0)
    plsc.subcore_barrier()
    pltpu.sync_copy(agg_sh.at[pl.ds(s * rows, rows)],
                    aggout.at[c, pl.ds(s * rows, rows)])
    if with_deg:
        pltpu.sync_copy(deg_sh.at[pl.ds(s * rows, rows)],
                        degout.at[c, pl.ds(s * rows, rows)])


def _sc_edge(xrel, gidx, dst, with_deg):
    e_pad = gidx.shape[0]
    nchunks = e_pad // (NW * EK)
    zrows = jnp.zeros((NPAD // NS, HID), jnp.float32)
    zdeg = jnp.zeros((NPAD // NS,), jnp.float32)
    onesrc = jnp.ones((EK,), jnp.float32)
    outs = [jax.ShapeDtypeStruct((NC, NPAD, HID), jnp.float32)]
    scratch = [
        pltpu.VMEM((EK,), jnp.int32),
        pltpu.VMEM((EK,), jnp.int32),
        pltpu.VMEM((EK, HID), jnp.float32),
        pltpu.VMEM((EK,), jnp.int32),
        pltpu.VMEM((EK,), jnp.int32),
        pltpu.VMEM((EK, HID), jnp.float32),
    ]
    if with_deg:
        outs.append(jax.ShapeDtypeStruct((NC, NPAD), jnp.float32))
        scratch.append(pltpu.VMEM((EK,), jnp.float32))
    scratch.append(pltpu.VMEM_SHARED((NPAD, HID), jnp.float32))
    if with_deg:
        scratch.append(pltpu.VMEM_SHARED((NPAD,), jnp.float32))
    scratch.append(pltpu.SemaphoreType.DMA)
    scratch.append(pltpu.SemaphoreType.DMA)
    k = functools.partial(
        pl.kernel,
        mesh=plsc.VectorSubcoreMesh(core_axis_name="c", subcore_axis_name="s"),
        out_type=tuple(outs) if with_deg else outs[0],
        scratch_types=scratch,
    )(functools.partial(_sc_edge_body, nchunks, with_deg))
    return k(xrel, gidx, dst, zrows, zdeg, onesrc)


def _tc_xrel_body(x_ref, bases_ref, coeffs_ref, out_ref):
    r = pl.program_id(0)
    sel = (lax.broadcasted_iota(jnp.int32, (R, 1), 0) == r).astype(jnp.float32)
    cvec = jnp.sum(coeffs_ref[...] * sel, axis=0)
    w = jnp.tensordot(cvec, bases_ref[...], axes=[[0], [0]])
    out_ref[0] = jnp.dot(x_ref[...], w, preferred_element_type=jnp.float32)


def _tc_xrel(x, bases, coeffs):
    blk = 512
    out = pl.pallas_call(
        _tc_xrel_body,
        grid=(R, NPAD // blk),
        in_specs=[
            pl.BlockSpec((blk, HID), lambda r, b: (b, 0)),
            pl.BlockSpec((NB, HID, HID), lambda r, b: (0, 0, 0)),
            pl.BlockSpec((R, NB), lambda r, b: (0, 0)),
        ],
        out_specs=pl.BlockSpec((1, blk, HID), lambda r, b: (r, b, 0)),
        out_shape=jax.ShapeDtypeStruct((R, NPAD, HID), jnp.float32),
    )(x, bases, coeffs)
    return out.reshape(R * NPAD, HID)


def _tc_combine_body(aggp_ref, degp_ref, x_ref, wself_ref, out_ref):
    agg = jnp.sum(aggp_ref[...], axis=0)
    deg = jnp.sum(degp_ref[...], axis=0)
    agg = agg / jnp.maximum(deg, 1.0)[:, None]
    out_ref[...] = jax.nn.relu(agg + jnp.dot(x_ref[...], wself_ref[...],
                                             preferred_element_type=jnp.float32))


def _tc_combine(aggp, degp, x, wself):
    blk = 512
    return pl.pallas_call(
        _tc_combine_body,
        grid=(NPAD // blk,),
        in_specs=[
            pl.BlockSpec((NC, blk, HID), lambda b: (0, b, 0)),
            pl.BlockSpec((NC, blk), lambda b: (0, b)),
            pl.BlockSpec((blk, HID), lambda b: (b, 0)),
            pl.BlockSpec((HID, HID), lambda b: (0, 0)),
        ],
        out_specs=pl.BlockSpec((blk, HID), lambda b: (b, 0)),
        out_shape=jax.ShapeDtypeStruct((NPAD, HID), jnp.float32),
    )(aggp, degp, x, wself)


def _tc_pool_body(h_ref, gid_ref, t_ref, out_ref, acc_ref):
    i = pl.program_id(0)
    nblocks = pl.num_programs(0)

    @pl.when(i == 0)
    def _():
        acc_ref[...] = jnp.full((32, HID), -1.0, jnp.float32)

    gid = gid_ref[...]
    hb = h_ref[...]
    for g in range(NG):
        m = gid == g
        vals = jnp.where(m, hb, -1.0)
        mx = jnp.max(vals, axis=0)
        acc_ref[g, :] = jnp.maximum(acc_ref[g, :], mx)

    @pl.when(i == nblocks - 1)
    def _():
        gi = acc_ref[...]
        gi = jnp.where(gi < 0.0, 0.0, gi)
        trep = t_ref[...]
        nrow = trep.shape[0]
        j2 = lax.broadcasted_iota(jnp.int32, (nrow, 1), 0) % SEQ
        start = jnp.maximum(trep - SEQ, 0)
        tsel = jnp.clip(start + j2, 0, NG - 1)
        msk = (j2 < (trep - start)).astype(jnp.float32)
        onehot = (lax.broadcasted_iota(jnp.int32, (nrow, 32), 1)
                  == tsel).astype(jnp.float32)
        res = jnp.dot(onehot, gi, preferred_element_type=jnp.float32)
        out_ref[...] = res * msk


def _tc_pool(h, gid_pad, t_rep):
    blk = 512
    nrow = t_rep.shape[0]
    return pl.pallas_call(
        _tc_pool_body,
        grid=(NPAD // blk,),
        in_specs=[
            pl.BlockSpec((blk, HID), lambda i: (i, 0)),
            pl.BlockSpec((blk, 1), lambda i: (i, 0)),
            pl.BlockSpec((nrow, 1), lambda i: (0, 0)),
        ],
        out_specs=pl.BlockSpec((nrow, HID), lambda i: (0, 0)),
        out_shape=jax.ShapeDtypeStruct((nrow, HID), jnp.float32),
        scratch_shapes=[pltpu.VMEM((32, HID), jnp.float32)],
    )(h, gid_pad, t_rep)


def kernel(t_list, ent_embeds, edge_index, edge_type, node_ids, node_graph_ids,
           bases1, coeffs1, wself1, bases2, coeffs2, wself2):
    n = node_ids.shape[0]
    e = edge_type.shape[0]
    nid_pad = jnp.concatenate(
        [node_ids, jnp.zeros((NPAD - n,), jnp.int32)])
    gid_pad = jnp.concatenate(
        [node_graph_ids, jnp.full((NPAD - n,), 31, jnp.int32)])[:, None]
    src2 = jnp.concatenate([edge_index[0], edge_index[1]])
    dst2 = jnp.concatenate([edge_index[1], edge_index[0]])
    et2 = jnp.concatenate([edge_type, edge_type])
    e2 = 2 * e
    grain = 2 * NW * EK
    e_pad = ((e2 + grain - 1) // grain) * grain
    gidx = et2 * NPAD + src2
    gidx = jnp.concatenate([gidx, jnp.zeros((e_pad - e2,), jnp.int32)])
    dstp = jnp.concatenate(
        [dst2, jnp.full((e_pad - e2,), n, jnp.int32)])

    x = _sc_gather(ent_embeds, nid_pad)
    xrel1 = _tc_xrel(x, bases1, coeffs1)
    aggp1, degp = _sc_edge(xrel1, gidx, dstp, with_deg=True)
    h1 = _tc_combine(aggp1, degp, x, wself1)
    xrel2 = _tc_xrel(h1, bases2, coeffs2)
    aggp2 = _sc_edge(xrel2, gidx, dstp, with_deg=False)
    h2 = _tc_combine(aggp2, degp, h1, wself2)
    emb = _tc_pool(h2, gid_pad, jnp.repeat(t_list, SEQ)[:, None])
    return emb.reshape(t_list.shape[0], SEQ, HID)
